# baseline (speedup 1.0000x reference)
import jax
import jax.numpy as jnp
from jax.experimental import pallas as pl
from jax.experimental.pallas import tpu as pltpu

_EPS = 1e-12
_BF = jnp.bfloat16


def _f32dot(a, b):
    return jnp.dot(a, b, preferred_element_type=jnp.float32)


def _make_kernel(nblk, nb, nb2):
    rs = nb2 // nb

    def _kernel(y_ref, h_ref, wc_ref, w1q_ref, w1o_ref, b1_ref, w2_ref,
                b2_ref, w3_ref, b3_ref, out_ref,
                hb_s, q_s, dvb_s, msgt_s, de_s, msgc_s, xn_s):
        i = pl.program_id(0)
        nbk, m = h_ref.shape
        K = y_ref.shape[1]

        @pl.when(i < nblk)
        def _phase1():
            hb = h_ref[...].astype(_BF)
            hb_s[pl.ds(i * nb, nb), :] = hb
            dv = _f32dot(hb, jnp.ones((m, 8), _BF))[:, :1]
            pde = _f32dot(jnp.ones((8, nb), _BF), hb)[:1, :]
            dvis = jax.lax.rsqrt(jnp.clip(dv, _EPS, None))
            dvb_s[pl.ds(i * nb, nb), :] = jnp.broadcast_to(
                dvis, (nb, K)).astype(_BF)
            q = jax.nn.softmax(y_ref[...], axis=-1)
            q_s[pl.ds(i * nb, nb), :] = q.astype(_BF)
            xn_s[pl.ds((i % rs) * nb, nb), :] = (q * dvis).astype(_BF)

            @pl.when(i == 0)
            def _init_de():
                de_s[...] = pde

            @pl.when(i != 0)
            def _acc_de():
                de_s[...] += pde

            @pl.when(i % rs == rs - 1)
            def _strip_dot():
                strip = i // rs
                pmsgt = jax.lax.dot_general(
                    xn_s[...],
                    hb_s[pl.ds(strip * nb2, nb2), :],
                    (((0,), (0,)), ((), ())),
                    preferred_element_type=jnp.float32)

                @pl.when(strip == 0)
                def _init():
                    msgt_s[...] = pmsgt

                @pl.when(strip != 0)
                def _acc():
                    msgt_s[...] += pmsgt

            @pl.when(i == nblk - 1)
            def _finalize():
                inv_de = 1.0 / jnp.clip(de_s[...], _EPS, None)
                msgn = jnp.transpose(msgt_s[...] * inv_de)
                msgc_s[...] = _f32dot(msgn.astype(_BF),
                                      wc_ref[...]).astype(_BF)

        @pl.when(i >= nblk)
        def _phase2():
            j = i - nblk
            qb = q_s[pl.ds(j * nb2, nb2), :]
            g = _f32dot(hb_s[pl.ds(j * nb2, nb2), :], msgc_s[...])
            gb = g.astype(_BF) * dvb_s[pl.ds(j * nb2, nb2), :]
            pre1 = (_f32dot(qb, w1q_ref[...])
                    + _f32dot(gb, w1o_ref[...])
                    + b1_ref[...])
            h1 = jnp.tanh(pre1)
            h2 = jnp.tanh(_f32dot(h1.astype(_BF), w2_ref[...]) + b2_ref[...])
            log_p = _f32dot(h2.astype(_BF), w3_ref[...]) + b3_ref[...]
            log_q = jnp.log(jnp.clip(qb.astype(jnp.float32), _EPS, None))
            drift = log_p - log_q
            out_ref[...] = drift - jnp.mean(drift, axis=-1, keepdims=True)

    return _kernel


def _row_block(n):
    for nb in (400, 512, 256, 128, 80, 16):
        if n % nb == 0 and nb % 16 == 0:
            return nb
    return n


def _row_block2(n):
    for nb2 in (2000, 2048, 1024, 512, 400, 256, 128, 80, 16):
        if n % nb2 == 0 and nb2 % 16 == 0:
            return nb2
    return n


def kernel(t, y, incidence, Wc, W1, b1, W2, b2, W3, b3):
    del t
    n, K = y.shape
    m = incidence.shape[1]
    obs_dim = Wc.shape[1]
    width = W1.shape[0]
    nb = _row_block(n)
    nblk = n // nb
    nb2 = _row_block2(n)
    nblk2 = n // nb2

    wcb = Wc.astype(_BF)
    w1q = W1[:, :K].T.astype(_BF)
    w1o = W1[:, K:].T.astype(_BF)
    w2t = W2.T.astype(_BF)
    w3t = W3.T.astype(_BF)
    b1r = b1.reshape(1, width)
    b2r = b2.reshape(1, width)
    b3r = b3.reshape(1, K)

    full = lambda r, c: pl.BlockSpec((r, c), lambda i: (0, 0))
    drift = pl.pallas_call(
        _make_kernel(nblk, nb, nb2),
        grid=(nblk + nblk2,),
        in_specs=[
            pl.BlockSpec((nb, K), lambda i: (jnp.minimum(i, nblk - 1), 0)),
            pl.BlockSpec((nb, m), lambda i: (jnp.minimum(i, nblk - 1), 0)),
            full(K, obs_dim),
            full(K, width),
            full(obs_dim, width),
            full(1, width),
            full(width, width),
            full(1, width),
            full(width, K),
            full(1, K),
        ],
        out_specs=pl.BlockSpec(
            (nb2, K), lambda i: (jnp.where(i < nblk, 0, i - nblk), 0)),
        out_shape=jax.ShapeDtypeStruct((n, K), jnp.float32),
        scratch_shapes=[
            pltpu.VMEM((n, m), _BF),
            pltpu.VMEM((n, K), _BF),
            pltpu.VMEM((n, K), _BF),
            pltpu.VMEM((K, m), jnp.float32),
            pltpu.VMEM((1, m), jnp.float32),
            pltpu.VMEM((m, obs_dim), _BF),
            pltpu.VMEM((nb2, K), _BF),
        ],
    )(y, incidence, wcb, w1q, w1o, b1r, w2t, b2r, w3t, b3r)
    return drift


# speedup vs baseline: 1.1209x; 1.1209x over previous
import jax
import jax.numpy as jnp
from jax.experimental import pallas as pl
from jax.experimental.pallas import tpu as pltpu

_EPS = 1e-12
_BF = jnp.bfloat16


def _f32dot(a, b):
    return jnp.dot(a, b, preferred_element_type=jnp.float32)


def _make_kernel(nblk, nb, nb2):
    rs = nb2 // nb
    nblk2 = nblk // rs

    def _kernel(y_ref, h_ref, wc_ref, w1q_ref, w1o_ref, b1_ref, w2_ref,
                b2_ref, w3_ref, b3_ref, out_ref,
                hb_s, q_s, msgt_s, de_s, msgc_s, g_s):
        i = pl.program_id(0)

        @pl.when(i < nblk)
        def _phase1():
            h = h_ref[...]
            dv = jnp.sum(h, axis=1, keepdims=True)
            pde = jnp.sum(h, axis=0, keepdims=True)
            dvis = jax.lax.rsqrt(jnp.clip(dv, _EPS, None))
            hb_s[pl.ds(i * nb, nb), :] = (h * dvis).astype(_BF)
            q = jax.nn.softmax(y_ref[...], axis=-1)
            q_s[pl.ds(i * nb, nb), :] = q.astype(_BF)

            @pl.when(i == 0)
            def _init_de():
                de_s[...] = pde

            @pl.when(i != 0)
            def _acc_de():
                de_s[...] += pde

            @pl.when(i % rs == rs - 1)
            def _strip_dot():
                strip = i // rs
                pmsgt = jax.lax.dot_general(
                    q_s[pl.ds(strip * nb2, nb2), :],
                    hb_s[pl.ds(strip * nb2, nb2), :],
                    (((0,), (0,)), ((), ())),
                    preferred_element_type=jnp.float32)

                @pl.when(strip == 0)
                def _init():
                    msgt_s[...] = pmsgt

                @pl.when(strip != 0)
                def _acc():
                    msgt_s[...] += pmsgt

            @pl.when(i == nblk - 1)
            def _finalize():
                inv_de = 1.0 / jnp.clip(de_s[...], _EPS, None)
                msgn = jnp.transpose(msgt_s[...] * inv_de)
                msgc_s[...] = _f32dot(msgn.astype(_BF),
                                      wc_ref[...]).astype(_BF)

        @pl.when(i >= nblk)
        def _phase2():
            j = i - nblk

            @pl.when(j < nblk2)
            def _gdot():
                g_s[pl.ds((j % 2) * nb2, nb2), :] = _f32dot(
                    hb_s[pl.ds(j * nb2, nb2), :], msgc_s[...]).astype(_BF)

            @pl.when(j >= 1)
            def _mlp():
                jm = j - 1
                qb = q_s[pl.ds(jm * nb2, nb2), :]
                gb = g_s[pl.ds((jm % 2) * nb2, nb2), :]
                pre1 = (_f32dot(qb, w1q_ref[...])
                        + _f32dot(gb, w1o_ref[...])
                        + b1_ref[...])
                h1 = jnp.tanh(pre1)
                h2 = jnp.tanh(_f32dot(h1.astype(_BF), w2_ref[...])
                              + b2_ref[...])
                log_p = _f32dot(h2.astype(_BF), w3_ref[...]) + b3_ref[...]
                log_q = jnp.log(jnp.clip(qb.astype(jnp.float32), _EPS, None))
                drift = log_p - log_q
                out_ref[...] = drift - jnp.mean(drift, axis=-1, keepdims=True)

    return _kernel


def _row_block(n):
    for nb in (400, 512, 256, 128, 80, 16):
        if n % nb == 0 and nb % 16 == 0:
            return nb
    return n


def _row_block2(n):
    for nb2 in (2000, 2048, 1024, 512, 400, 256, 128, 80, 16):
        if n % nb2 == 0 and nb2 % 16 == 0:
            return nb2
    return n


def kernel(t, y, incidence, Wc, W1, b1, W2, b2, W3, b3):
    del t
    n, K = y.shape
    m = incidence.shape[1]
    obs_dim = Wc.shape[1]
    width = W1.shape[0]
    nb = _row_block(n)
    nblk = n // nb
    nb2 = _row_block2(n)
    nblk2 = n // nb2

    wcb = Wc.astype(_BF)
    w1q = W1[:, :K].T.astype(_BF)
    w1o = W1[:, K:].T.astype(_BF)
    w2t = W2.T.astype(_BF)
    w3t = W3.T.astype(_BF)
    b1r = b1.reshape(1, width)
    b2r = b2.reshape(1, width)
    b3r = b3.reshape(1, K)

    full = lambda r, c: pl.BlockSpec((r, c), lambda i: (0, 0))
    drift = pl.pallas_call(
        _make_kernel(nblk, nb, nb2),
        grid=(nblk + nblk2 + 1,),
        in_specs=[
            pl.BlockSpec((nb, K), lambda i: (jnp.minimum(i, nblk - 1), 0)),
            pl.BlockSpec((nb, m), lambda i: (jnp.minimum(i, nblk - 1), 0)),
            full(K, obs_dim),
            full(K, width),
            full(obs_dim, width),
            full(1, width),
            full(width, width),
            full(1, width),
            full(width, K),
            full(1, K),
        ],
        out_specs=pl.BlockSpec(
            (nb2, K), lambda i: (jnp.where(i <= nblk, 0, i - nblk - 1), 0)),
        out_shape=jax.ShapeDtypeStruct((n, K), jnp.float32),
        scratch_shapes=[
            pltpu.VMEM((n, m), _BF),
            pltpu.VMEM((n, K), _BF),
            pltpu.VMEM((K, m), jnp.float32),
            pltpu.VMEM((1, m), jnp.float32),
            pltpu.VMEM((m, obs_dim), _BF),
            pltpu.VMEM((2 * nb2, obs_dim), _BF),
        ],
    )(y, incidence, wcb, w1q, w1o, b1r, w2t, b2r, w3t, b3r)
    return drift


# speedup vs baseline: 1.1385x; 1.0157x over previous
import jax
import jax.numpy as jnp
from jax.experimental import pallas as pl
from jax.experimental.pallas import tpu as pltpu

_EPS = 1e-12
_BF = jnp.bfloat16


def _f32dot(a, b):
    return jnp.dot(a, b, preferred_element_type=jnp.float32)


def _make_kernel(nblk, nb, nb2):
    rs = nb2 // nb
    nblk2 = nblk // rs

    def _kernel(y_ref, h_ref, wc_ref, w1q_ref, w1o_ref, b1_ref, w2_ref,
                b2_ref, w3_ref, b3_ref, out_ref,
                hb_s, q_s, dvb_s, msgt_s, de_s, msgc_s, g_s, xn_s):
        i = pl.program_id(0)
        K = y_ref.shape[1]

        @pl.when(i < nblk)
        def _phase1():
            h = h_ref[...]
            dv = jnp.sum(h, axis=1, keepdims=True)
            pde = jnp.sum(h, axis=0, keepdims=True)
            dvis = jax.lax.rsqrt(jnp.clip(dv, _EPS, None))
            hb_s[pl.ds(i * nb, nb), :] = h.astype(_BF)
            dvb_s[pl.ds(i * nb, nb), :] = jnp.broadcast_to(
                dvis, (nb, K)).astype(_BF)
            q = jax.nn.softmax(y_ref[...], axis=-1)
            q_s[pl.ds(i * nb, nb), :] = q.astype(_BF)
            xn_s[pl.ds((i % rs) * nb, nb), :] = (q * dvis).astype(_BF)

            @pl.when(i == 0)
            def _init_de():
                de_s[...] = pde

            @pl.when(i != 0)
            def _acc_de():
                de_s[...] += pde

            @pl.when(i % rs == rs - 1)
            def _strip_dot():
                strip = i // rs
                pmsgt = jax.lax.dot_general(
                    xn_s[...],
                    hb_s[pl.ds(strip * nb2, nb2), :],
                    (((0,), (0,)), ((), ())),
                    preferred_element_type=jnp.float32)

                @pl.when(strip == 0)
                def _init():
                    msgt_s[...] = pmsgt

                @pl.when(strip != 0)
                def _acc():
                    msgt_s[...] += pmsgt

            @pl.when(i == nblk - 1)
            def _finalize():
                inv_de = 1.0 / jnp.clip(de_s[...], _EPS, None)
                msgn = jnp.transpose(msgt_s[...] * inv_de)
                msgc_s[...] = _f32dot(msgn.astype(_BF),
                                      wc_ref[...]).astype(_BF)

        @pl.when(i >= nblk)
        def _phase2():
            j = i - nblk

            @pl.when(j < nblk2)
            def _gdot():
                g_s[pl.ds((j % 2) * nb2, nb2), :] = (
                    _f32dot(hb_s[pl.ds(j * nb2, nb2), :],
                            msgc_s[...]).astype(_BF)
                    * dvb_s[pl.ds(j * nb2, nb2), :])

            @pl.when(j >= 1)
            def _mlp():
                jm = j - 1
                qb = q_s[pl.ds(jm * nb2, nb2), :]
                gb = g_s[pl.ds((jm % 2) * nb2, nb2), :]
                pre1 = (_f32dot(qb, w1q_ref[...])
                        + _f32dot(gb, w1o_ref[...])
                        + b1_ref[...])
                h1 = jnp.tanh(pre1)
                h2 = jnp.tanh(_f32dot(h1.astype(_BF), w2_ref[...])
                              + b2_ref[...])
                log_p = _f32dot(h2.astype(_BF), w3_ref[...]) + b3_ref[...]
                log_q = jnp.log(jnp.clip(qb.astype(jnp.float32), _EPS, None))
                drift = log_p - log_q
                out_ref[...] = drift - jnp.mean(drift, axis=-1, keepdims=True)

    return _kernel


def _row_block(n):
    for nb in (400, 512, 256, 128, 80, 16):
        if n % nb == 0 and nb % 16 == 0:
            return nb
    return n


def _row_block2(n):
    for nb2 in (2000, 2048, 1024, 512, 400, 256, 128, 80, 16):
        if n % nb2 == 0 and nb2 % 16 == 0:
            return nb2
    return n


def kernel(t, y, incidence, Wc, W1, b1, W2, b2, W3, b3):
    del t
    n, K = y.shape
    m = incidence.shape[1]
    obs_dim = Wc.shape[1]
    width = W1.shape[0]
    nb = _row_block(n)
    nblk = n // nb
    nb2 = _row_block2(n)
    nblk2 = n // nb2

    wcb = Wc.astype(_BF)
    w1q = W1[:, :K].T.astype(_BF)
    w1o = W1[:, K:].T.astype(_BF)
    w2t = W2.T.astype(_BF)
    w3t = W3.T.astype(_BF)
    b1r = b1.reshape(1, width)
    b2r = b2.reshape(1, width)
    b3r = b3.reshape(1, K)

    full = lambda r, c: pl.BlockSpec((r, c), lambda i: (0, 0))
    drift = pl.pallas_call(
        _make_kernel(nblk, nb, nb2),
        grid=(nblk + nblk2 + 1,),
        in_specs=[
            pl.BlockSpec((nb, K), lambda i: (jnp.minimum(i, nblk - 1), 0)),
            pl.BlockSpec((nb, m), lambda i: (jnp.minimum(i, nblk - 1), 0)),
            full(K, obs_dim),
            full(K, width),
            full(obs_dim, width),
            full(1, width),
            full(width, width),
            full(1, width),
            full(width, K),
            full(1, K),
        ],
        out_specs=pl.BlockSpec(
            (nb2, K), lambda i: (jnp.where(i <= nblk, 0, i - nblk - 1), 0)),
        out_shape=jax.ShapeDtypeStruct((n, K), jnp.float32),
        scratch_shapes=[
            pltpu.VMEM((n, m), _BF),
            pltpu.VMEM((n, K), _BF),
            pltpu.VMEM((n, K), _BF),
            pltpu.VMEM((K, m), jnp.float32),
            pltpu.VMEM((1, m), jnp.float32),
            pltpu.VMEM((m, obs_dim), _BF),
            pltpu.VMEM((2 * nb2, obs_dim), _BF),
            pltpu.VMEM((nb2, K), _BF),
        ],
    )(y, incidence, wcb, w1q, w1o, b1r, w2t, b2r, w3t, b3r)
    return drift
